# Initial kernel scaffold; baseline (speedup 1.0000x reference)
import jax
import jax.numpy as jnp
from jax.experimental import pallas as pl


def kernel(question, question_mask, img_feat, img_loc, img_node1_id_list, img_node2_id_list, kg_entity, kg_edge, kg_node1_ids_list, kg_node2_ids_list, params):
    raise NotImplementedError("write your pallas kernel here")


# speedup vs baseline: 7.1746x; 7.1746x over previous
import jax
import jax.numpy as jnp
from jax.experimental import pallas as pl
from jax.experimental.pallas import tpu as pltpu

B = 32; L = 20; H = 1024; EMB = 300; VOCAB = 10000; NA = 3000
IMG = 2048; LOC = 5; NO = 36; EI = 1260; NK = 100; EK = 200; LK = 10

f32 = jnp.float32


def _dotT(a, b):
    return jax.lax.dot_general(a, b, (((1,), (1,)), ((), ())),
                               preferred_element_type=f32)


def _mm(a, b):
    return jnp.dot(a, b, preferred_element_type=f32)


def _lstm_body(emb_ref, wx_ref, wh_ref, b_ref, allout_ref, hlast_ref, x_scr):
    x_scr[...] = _mm(emb_ref[...], wx_ref[...]) + b_ref[...]

    def step(t, carry):
        h, c = carry
        z = x_scr[pl.ds(t * B, B), :] + _mm(h, wh_ref[...])
        i = z[:, :H]; f = z[:, H:2 * H]; g = z[:, 2 * H:3 * H]; o = z[:, 3 * H:]
        c = jax.nn.sigmoid(f) * c + jax.nn.sigmoid(i) * jnp.tanh(g)
        h = jax.nn.sigmoid(o) * jnp.tanh(c)
        allout_ref[pl.ds(t * B, B), :] = h
        return (h, c)

    z0 = jnp.zeros((B, H), f32)
    h, _ = jax.lax.fori_loop(0, L, step, (z0, z0))
    hlast_ref[...] = h


def _run_lstm(emb_t, wx, wh, b):
    return pl.pallas_call(
        _lstm_body,
        out_shape=(jax.ShapeDtypeStruct((L * B, H), f32),
                   jax.ShapeDtypeStruct((B, H), f32)),
        scratch_shapes=[pltpu.VMEM((L * B, 4 * H), f32)],
    )(emb_t, wx, wh, b)


def _qpool_body(ao_ref, h_ref, m_ref, qiw_ref, qib_ref, qkw_ref, qkb_ref,
                w1i_ref, w2i_ref, bi_ref, w1k_ref, w2k_ref, bk_ref,
                qimg_ref, qkg_ref):
    ao = ao_ref[...]
    h = h_ref[...]
    mask = m_ref[...]

    def pool(qw, qb, w1, w2, ab):
        q = _mm(h, qw) + qb
        s = (jnp.sum(ao * w1[None], axis=-1, keepdims=True)
             + jnp.sum(q * w2, axis=-1, keepdims=True)[None]
             + ab[None])
        s = jnp.where(mask == 1, -1e32, s)
        mx = jnp.max(s, axis=0, keepdims=True)
        ex = jnp.exp(s - mx)
        a = ex / jnp.sum(ex, axis=0, keepdims=True)
        return jnp.sum(ao * a, axis=0)

    qimg_ref[...] = pool(qiw_ref[...], qib_ref[...], w1i_ref[...],
                         w2i_ref[...], bi_ref[...])
    qkg_ref[...] = pool(qkw_ref[...], qkb_ref[...], w1k_ref[...],
                        w2k_ref[...], bk_ref[...])


def _run_qpool(ao3, h, mask_t, qiw, qib, qkw, qkb, w1i, w2i, bi, w1k, w2k, bk):
    return pl.pallas_call(
        _qpool_body,
        out_shape=(jax.ShapeDtypeStruct((B, H), f32),
                   jax.ShapeDtypeStruct((B, H), f32)),
    )(ao3, h, mask_t, qiw, qib, qkw, qkb, w1i, w2i, bi, w1k, w2k, bk)


def _inode_body(feat_ref, w_ref, b_ref, loc_ref, ew_ref, eb_ref,
                wa1_ref, wa2_ref, nodes_ref, e1_ref, e2_ref):
    nodes_ref[...] = _mm(feat_ref[...], w_ref[...]) + b_ref[...]
    loc = loc_ref[...]

    def eatt(wa):
        fold = _dotT(ew_ref[...], wa)
        c = _dotT(eb_ref[...], wa)
        return jnp.sum(loc * fold, axis=0, keepdims=True) + c

    e1_ref[...] = eatt(wa1_ref[...])
    e2_ref[...] = eatt(wa2_ref[...])


def _run_inode(feat, w, b, loc_t, ew, eb, wa1, wa2):
    return pl.pallas_call(
        _inode_body,
        out_shape=(jax.ShapeDtypeStruct((B * NO, H), f32),
                   jax.ShapeDtypeStruct((1, B * EI), f32),
                   jax.ShapeDtypeStruct((1, B * EI), f32)),
    )(feat, w, b, loc_t, ew, eb, wa1, wa2)


def _kgfeat_body(ent_ref, eid_ref, edg_ref, gid_ref, nw_ref, nb_ref,
                 ew_ref, eb_ref, wa1_ref, wa2_ref,
                 nodes_ref, e1_ref, e2_ref):
    ent = ent_ref[...].reshape(NK, LK, EMB)
    eid = eid_ref[...].reshape(NK, LK)
    elen = jnp.maximum(jnp.sum((eid != 1.0).astype(f32), axis=-1,
                               keepdims=True), 1.0)
    efeat = jnp.sum(ent, axis=1) / elen
    nodes_ref[...] = (_mm(efeat, nw_ref[...]) + nb_ref[...])[None]

    edg = edg_ref[...].reshape(EK, LK, EMB)
    gid = gid_ref[...].reshape(EK, LK)
    glen = jnp.maximum(jnp.sum((gid != 1.0).astype(f32), axis=-1,
                               keepdims=True), 1.0)
    gfeat = jnp.sum(edg, axis=1) / glen

    def eatt(wa):
        fold = _dotT(ew_ref[...], wa)
        c = _dotT(eb_ref[...], wa)
        return _mm(gfeat, fold) + c

    e1_ref[...] = eatt(wa1_ref[...])[None]
    e2_ref[...] = eatt(wa2_ref[...])[None]


def _run_kgfeat(ent, eid, edg, gid, nw, nb, ew, eb, wa1, wa2):
    spec = lambda shape: pl.BlockSpec((1,) + shape, lambda i: (i, 0, 0))
    full = lambda a: pl.BlockSpec(a.shape, lambda i: (0,) * a.ndim)
    return pl.pallas_call(
        _kgfeat_body,
        grid=(B,),
        in_specs=[spec((NK * LK, EMB)), spec((NK, LK)),
                  spec((EK * LK, EMB)), spec((EK, LK)),
                  full(nw), full(nb), full(ew), full(eb), full(wa1), full(wa2)],
        out_specs=(spec((NK, H)), spec((EK, 1)), spec((EK, 1))),
        out_shape=(jax.ShapeDtypeStruct((B, NK, H), f32),
                   jax.ShapeDtypeStruct((B, EK, 1), f32),
                   jax.ShapeDtypeStruct((B, EK, 1), f32)),
    )(ent, eid, edg, gid, nw, nb, ew, eb, wa1, wa2)


def _reason_body(nodes_ref, eatt_ref, src_ref, dst_ref, q_ref, wn_ref, wg_ref,
                 aggn_ref, *, n, e):
    nodes = nodes_ref[...].reshape(n, H)
    ea = eatt_ref[...].reshape(1, e)
    src = src_ref[...].reshape(1, e)
    dst = dst_ref[...].reshape(1, e)
    q = q_ref[...].reshape(1, H)
    n_att = _dotT(nodes, wn_ref[...])
    q_att = _dotT(q, wg_ref[...])
    rows = jax.lax.broadcasted_iota(jnp.int32, (n, e), 0)
    oh_src = (rows == src).astype(f32)
    oh_dst = (rows == dst).astype(f32)
    gat = jnp.sum(oh_src * n_att, axis=0, keepdims=True)
    s = jnp.tanh(gat + ea + q_att)
    mx = jnp.max(jnp.where(oh_dst > 0.5, s, -1e30), axis=1, keepdims=True)
    m_e = jnp.sum(oh_dst * mx, axis=0, keepdims=True)
    ex = jnp.exp(s - m_e)
    den = jnp.sum(oh_dst * ex, axis=1, keepdims=True)
    den_e = jnp.sum(oh_dst * den, axis=0, keepdims=True)
    alpha = ex / (den_e + 1e-9)
    adj = _dotT(oh_dst * alpha, oh_src)
    aggn_ref[...] = _mm(adj, nodes)[None]


def _run_reason(nodes3, eatt, src, dst, q3, wn, wg, n, e):
    import functools
    spec = lambda shape: pl.BlockSpec((1,) + shape, lambda i: (i, 0, 0))
    full = lambda a: pl.BlockSpec(a.shape, lambda i: (0,) * a.ndim)
    return pl.pallas_call(
        functools.partial(_reason_body, n=n, e=e),
        grid=(B,),
        in_specs=[spec((n, H)), spec((1, e)), spec((1, e)), spec((1, e)),
                  spec((1, H)), full(wn), full(wg)],
        out_specs=spec((n, H)),
        out_shape=jax.ShapeDtypeStruct((B, n, H), f32),
    )(nodes3, eatt, src, dst, q3, wn, wg)


def _post_body(nodes_ref, aggn_ref, wm_ref, aw_ref, v_ref, q_ref,
               nn_ref, out_ref, *, n, gb):
    msg = _mm(aggn_ref[...], wm_ref[...])
    nn = jax.nn.relu(nodes_ref[...] + msg)
    nn_ref[...] = nn
    a1 = _mm(nn, aw_ref[:H, :])
    qw = _mm(q_ref[...], aw_ref[H:, :])
    t3 = jnp.tanh(a1.reshape(gb, n, H) + qw[:, None, :])
    sc = jnp.sum(t3 * v_ref[...][None], axis=-1, keepdims=True)
    mx = jnp.max(sc, axis=1, keepdims=True)
    ex = jnp.exp(sc - mx)
    a = ex / jnp.sum(ex, axis=1, keepdims=True)
    out_ref[...] = jnp.sum(nn.reshape(gb, n, H) * a, axis=1)


def _run_post(nodes, aggn, wm, aw, vrow, q, n):
    import functools
    gb = 8
    grid = B // gb
    rb = gb * n
    rows = lambda: pl.BlockSpec((rb, H), lambda i: (i, 0))
    full = lambda a: pl.BlockSpec(a.shape, lambda i: (0,) * a.ndim)
    return pl.pallas_call(
        functools.partial(_post_body, n=n, gb=gb),
        grid=(grid,),
        in_specs=[rows(), rows(), full(wm), full(aw), full(vrow),
                  pl.BlockSpec((gb, H), lambda i: (i, 0))],
        out_specs=(rows(), pl.BlockSpec((gb, H), lambda i: (i, 0))),
        out_shape=(jax.ShapeDtypeStruct((B * n, H), f32),
                   jax.ShapeDtypeStruct((B, H), f32)),
    )(nodes, aggn, wm, aw, vrow, q)


def _rel_body(qi_ref, qk_ref, io_ref, ko_ref, iw_ref, ib_ref, kw_ref, kb_ref,
              ir_ref, kr_ref):
    iw = iw_ref[...]; kw = kw_ref[...]
    ir_ref[...] = (_mm(qi_ref[...], iw[:H, :]) + _mm(ko_ref[...], iw[H:, :])
                   + ib_ref[...])
    kr_ref[...] = (_mm(qk_ref[...], kw[:H, :]) + _mm(io_ref[...], kw[H:, :])
                   + kb_ref[...])


def _run_rel(qi, qk, io1, ko1, iw, ib, kw, kb):
    return pl.pallas_call(
        _rel_body,
        out_shape=(jax.ShapeDtypeStruct((B, H), f32),
                   jax.ShapeDtypeStruct((B, H), f32)),
    )(qi, qk, io1, ko1, iw, ib, kw, kb)


def _head_body(io1_ref, io2_ref, ko1_ref, ko2_ref, gi_ref, gk_ref, pg_ref,
               hw_ref, out_ref):
    img_vec = io1_ref[...] + io2_ref[...]
    kg_vec = ko1_ref[...] + ko2_ref[...]
    gate = jax.nn.sigmoid(_mm(img_vec, gi_ref[...]) + _mm(kg_vec, gk_ref[...]))
    fused = gate * img_vec + (1.0 - gate) * kg_vec
    fused = fused * jax.nn.sigmoid(_mm(fused, pg_ref[...]))
    out_ref[...] = _mm(fused, hw_ref[...])


def _run_head(io1, io2, ko1, ko2, gi, gk, pg, hw):
    return pl.pallas_call(
        _head_body,
        out_shape=jax.ShapeDtypeStruct((B, NA), f32),
    )(io1, io2, ko1, ko2, gi, gk, pg, hw)


def kernel(question, question_mask, img_feat, img_loc, img_node1_id_list,
           img_node2_id_list, kg_entity, kg_edge, kg_node1_ids_list,
           kg_node2_ids_list, params):
    p = params
    row = lambda a: a.reshape(1, -1).astype(f32)

    emb = p['word_emb'][question]
    emb_t = jnp.swapaxes(emb, 0, 1).reshape(L * B, EMB)
    all_out, h_last = _run_lstm(emb_t, p['lstm_Wx'], p['lstm_Wh'],
                                row(p['lstm_b']))
    ao3 = all_out.reshape(L, B, H)
    mask_t = jnp.swapaxes(question_mask, 0, 1)[:, :, None]
    ques_img, ques_kg = _run_qpool(
        ao3, h_last, mask_t,
        p['qi_W'], row(p['qi_b']), p['qk_W'], row(p['qk_b']),
        row(p['qia_W'][:H, 0]), row(p['qia_W'][H:, 0]), row(p['qia_b']),
        row(p['qka_W'][:H, 0]), row(p['qka_W'][H:, 0]), row(p['qka_b']))

    loc_t = jnp.swapaxes(img_loc.reshape(B * EI, LOC), 0, 1)
    img_nodes, ie1, ie2 = _run_inode(
        img_feat.reshape(B * NO, IMG), p['inode_W'], row(p['inode_b']),
        loc_t, p['iedge_W'], row(p['iedge_b']),
        row(p['img_att_W'][H:2 * H, 0]), row(p['imgx_att_W'][H:2 * H, 0]))
    ie1 = ie1.reshape(B, 1, EI)
    ie2 = ie2.reshape(B, 1, EI)

    ent_emb = p['word_emb'][kg_entity].reshape(B, NK * LK, EMB)
    edg_emb = p['word_emb'][kg_edge].reshape(B, EK * LK, EMB)
    kg_nodes3, ke1, ke2 = _run_kgfeat(
        ent_emb, kg_entity.reshape(B, NK, LK).astype(f32),
        edg_emb, kg_edge.reshape(B, EK, LK).astype(f32),
        p['knode_W'], row(p['knode_b']), p['kedge_W'], row(p['kedge_b']),
        row(p['kg_att_W'][H:2 * H, 0]), row(p['kgx_att_W'][H:2 * H, 0]))
    ke1 = ke1.reshape(B, 1, EK)
    ke2 = ke2.reshape(B, 1, EK)

    n1 = img_node1_id_list[:, None, :]
    n2 = img_node2_id_list[:, None, :]
    k1 = kg_node1_ids_list[:, None, :]
    k2 = kg_node2_ids_list[:, None, :]

    img_nodes3 = img_nodes.reshape(B, NO, H)
    aggn_i1 = _run_reason(img_nodes3, ie1, n1, n2, ques_img[:, None, :],
                          row(p['img_att_W'][:H, 0]),
                          row(p['img_att_W'][2 * H:, 0]), NO, EI)
    img_nodes1, img_out_1 = _run_post(
        img_nodes, aggn_i1.reshape(B * NO, H), p['img_msg_W'],
        p['img_agg_W'], row(p['img_agg_v'][:, 0]), ques_img, NO)

    aggn_k1 = _run_reason(kg_nodes3, ke1, k1, k2, ques_kg[:, None, :],
                          row(p['kg_att_W'][:H, 0]),
                          row(p['kg_att_W'][2 * H:, 0]), NK, EK)
    kg_nodes1, kg_out_1 = _run_post(
        kg_nodes3.reshape(B * NK, H), aggn_k1.reshape(B * NK, H),
        p['kg_msg_W'], p['kg_agg_W'], row(p['kg_agg_v'][:, 0]), ques_kg, NK)

    img_rel, kg_rel = _run_rel(ques_img, ques_kg, img_out_1, kg_out_1,
                               p['img_rel_W'], row(p['img_rel_b']),
                               p['kg_rel_W'], row(p['kg_rel_b']))

    aggn_i2 = _run_reason(img_nodes1.reshape(B, NO, H), ie2, n1, n2,
                          img_rel[:, None, :],
                          row(p['imgx_att_W'][:H, 0]),
                          row(p['imgx_att_W'][2 * H:, 0]), NO, EI)
    _, img_out_2 = _run_post(
        img_nodes1, aggn_i2.reshape(B * NO, H), p['imgx_msg_W'],
        p['imgx_agg_W'], row(p['imgx_agg_v'][:, 0]), img_rel, NO)

    aggn_k2 = _run_reason(kg_nodes1.reshape(B, NK, H), ke2, k1, k2,
                          kg_rel[:, None, :],
                          row(p['kgx_att_W'][:H, 0]),
                          row(p['kgx_att_W'][2 * H:, 0]), NK, EK)
    _, kg_out_2 = _run_post(
        kg_nodes1, aggn_k2.reshape(B * NK, H), p['kgx_msg_W'],
        p['kgx_agg_W'], row(p['kgx_agg_v'][:, 0]), kg_rel, NK)

    return _run_head(img_out_1, img_out_2, kg_out_1, kg_out_2,
                     p['img_gate_W'], p['kg_gate_W'], p['pred_gate_W'],
                     p['head_W'])


# speedup vs baseline: 8.7942x; 1.2257x over previous
import functools

import jax
import jax.numpy as jnp
from jax.experimental import pallas as pl
from jax.experimental.pallas import tpu as pltpu
from jax.experimental.pallas import tpu_sc as plsc

B = 32; L = 20; H = 1024; EMB = 300; VOCAB = 10000; NA = 3000
IMG = 2048; LOC = 5; NO = 36; EI = 1260; NK = 100; EK = 200; LK = 10

f32 = jnp.float32


def _dotT(a, b):
    return jax.lax.dot_general(a, b, (((1,), (1,)), ((), ())),
                               preferred_element_type=f32)


def _mm(a, b):
    return jnp.dot(a, b, preferred_element_type=f32)


_DP = 384
_GPW = 304
_CG = 16
_NCHUNK = _GPW // _CG


def _sc_gather_body(table_ref, ids_ref, out_ref, idx_v, rows_v, out_v, sem):
    wid = jax.lax.axis_index("s") * 2 + jax.lax.axis_index("c")

    def chunk(c, carry):
        g0 = wid * _GPW + c * _CG
        pltpu.sync_copy(ids_ref.at[pl.ds(g0 * LK, _CG * LK)], idx_v)
        pltpu.async_copy(table_ref.at[idx_v], rows_v, sem).wait()

        def group(g, carry2):
            for j in range(_DP // 16):
                sl = pl.ds(j * 16, 16)
                acc = rows_v[g * LK, sl]
                for r in range(1, LK):
                    acc = acc + rows_v[g * LK + r, sl]
                out_v[g, sl] = acc
            return carry2

        jax.lax.fori_loop(0, _CG, group, 0)
        pltpu.sync_copy(out_v, out_ref.at[pl.ds(g0, _CG)])
        return carry

    jax.lax.fori_loop(0, _NCHUNK, chunk, 0)


def _run_sc_gather(table_pad, ids_pad):
    k = pl.kernel(
        _sc_gather_body,
        mesh=plsc.VectorSubcoreMesh(core_axis_name="c", subcore_axis_name="s"),
        out_type=jax.ShapeDtypeStruct((32 * _GPW, _DP), f32),
        scratch_types=[
            pltpu.VMEM((_CG * LK,), jnp.int32),
            pltpu.VMEM((_CG * LK, _DP), f32),
            pltpu.VMEM((_CG, _DP), f32),
            pltpu.SemaphoreType.DMA,
        ],
    )
    return k(table_pad, ids_pad)


def _lstm_body(emb_ref, wx_ref, wh_ref, b_ref, allout_ref, hlast_ref, x_scr):
    x_scr[...] = _mm(emb_ref[...], wx_ref[...]) + b_ref[...]

    def step(t, carry):
        h, c = carry
        z = x_scr[pl.ds(t * B, B), :] + _mm(h, wh_ref[...])
        i = z[:, :H]; f = z[:, H:2 * H]; g = z[:, 2 * H:3 * H]; o = z[:, 3 * H:]
        c = jax.nn.sigmoid(f) * c + jax.nn.sigmoid(i) * jnp.tanh(g)
        h = jax.nn.sigmoid(o) * jnp.tanh(c)
        allout_ref[pl.ds(t * B, B), :] = h
        return (h, c)

    z0 = jnp.zeros((B, H), f32)
    h, _ = jax.lax.fori_loop(0, L, step, (z0, z0))
    hlast_ref[...] = h


def _run_lstm(emb_t, wx, wh, b):
    return pl.pallas_call(
        _lstm_body,
        out_shape=(jax.ShapeDtypeStruct((L * B, H), f32),
                   jax.ShapeDtypeStruct((B, H), f32)),
        scratch_shapes=[pltpu.VMEM((L * B, 4 * H), f32)],
    )(emb_t, wx, wh, b)


def _qpool_body(ao_ref, h_ref, m_ref, qiw_ref, qib_ref, qkw_ref, qkb_ref,
                w1i_ref, w2i_ref, bi_ref, w1k_ref, w2k_ref, bk_ref,
                qimg_ref, qkg_ref):
    ao = ao_ref[...]
    h = h_ref[...]
    mask = m_ref[...]

    def pool(qw, qb, w1, w2, ab):
        q = _mm(h, qw) + qb
        s = (jnp.sum(ao * w1[None], axis=-1, keepdims=True)
             + jnp.sum(q * w2, axis=-1, keepdims=True)[None]
             + ab[None])
        s = jnp.where(mask == 1, -1e32, s)
        mx = jnp.max(s, axis=0, keepdims=True)
        ex = jnp.exp(s - mx)
        a = ex / jnp.sum(ex, axis=0, keepdims=True)
        return jnp.sum(ao * a, axis=0)

    qimg_ref[...] = pool(qiw_ref[...], qib_ref[...], w1i_ref[...],
                         w2i_ref[...], bi_ref[...])
    qkg_ref[...] = pool(qkw_ref[...], qkb_ref[...], w1k_ref[...],
                        w2k_ref[...], bk_ref[...])


def _run_qpool(ao3, h, mask_t, qiw, qib, qkw, qkb, w1i, w2i, bi, w1k, w2k, bk):
    return pl.pallas_call(
        _qpool_body,
        out_shape=(jax.ShapeDtypeStruct((B, H), f32),
                   jax.ShapeDtypeStruct((B, H), f32)),
    )(ao3, h, mask_t, qiw, qib, qkw, qkb, w1i, w2i, bi, w1k, w2k, bk)


def _inode_body(feat_ref, w_ref, b_ref, loc_ref, ew_ref, eb_ref,
                wa1_ref, wa2_ref, nodes_ref, e1_ref, e2_ref):
    nodes_ref[...] = _mm(feat_ref[...], w_ref[...]) + b_ref[...]
    loc = loc_ref[...]

    def eatt(wa):
        fold = _dotT(ew_ref[...], wa)
        c = _dotT(eb_ref[...], wa)
        return jnp.sum(loc * fold, axis=0, keepdims=True) + c

    e1_ref[...] = eatt(wa1_ref[...])
    e2_ref[...] = eatt(wa2_ref[...])


def _run_inode(feat, w, b, loc_t, ew, eb, wa1, wa2):
    return pl.pallas_call(
        _inode_body,
        out_shape=(jax.ShapeDtypeStruct((B * NO, H), f32),
                   jax.ShapeDtypeStruct((1, B * EI), f32),
                   jax.ShapeDtypeStruct((1, B * EI), f32)),
    )(feat, w, b, loc_t, ew, eb, wa1, wa2)


def _kgfeat_body(ent_ref, eid_ref, edg_ref, gid_ref, nw_ref, nb_ref,
                 ew_ref, eb_ref, wa1_ref, wa2_ref,
                 nodes_ref, e1_ref, e2_ref):
    ent = ent_ref[...].reshape(NK, EMB)
    eid = eid_ref[...].reshape(NK, LK)
    elen = jnp.maximum(jnp.sum((eid != 1.0).astype(f32), axis=-1,
                               keepdims=True), 1.0)
    efeat = ent / elen
    nodes_ref[...] = (_mm(efeat, nw_ref[...]) + nb_ref[...])[None]

    edg = edg_ref[...].reshape(EK, EMB)
    gid = gid_ref[...].reshape(EK, LK)
    glen = jnp.maximum(jnp.sum((gid != 1.0).astype(f32), axis=-1,
                               keepdims=True), 1.0)
    gfeat = edg / glen

    def eatt(wa):
        fold = _dotT(ew_ref[...], wa)
        c = _dotT(eb_ref[...], wa)
        return _mm(gfeat, fold) + c

    e1_ref[...] = eatt(wa1_ref[...])[None]
    e2_ref[...] = eatt(wa2_ref[...])[None]


def _run_kgfeat(ent, eid, edg, gid, nw, nb, ew, eb, wa1, wa2):
    spec = lambda shape: pl.BlockSpec((1,) + shape, lambda i: (i, 0, 0))
    full = lambda a: pl.BlockSpec(a.shape, lambda i: (0,) * a.ndim)
    return pl.pallas_call(
        _kgfeat_body,
        grid=(B,),
        in_specs=[spec((NK, EMB)), spec((NK, LK)),
                  spec((EK, EMB)), spec((EK, LK)),
                  full(nw), full(nb), full(ew), full(eb), full(wa1), full(wa2)],
        out_specs=(spec((NK, H)), spec((EK, 1)), spec((EK, 1))),
        out_shape=(jax.ShapeDtypeStruct((B, NK, H), f32),
                   jax.ShapeDtypeStruct((B, EK, 1), f32),
                   jax.ShapeDtypeStruct((B, EK, 1), f32)),
    )(ent, eid, edg, gid, nw, nb, ew, eb, wa1, wa2)


def _reason_body(nodes_ref, eatt_ref, src_ref, dst_ref, q_ref, wn_ref, wg_ref,
                 aggn_ref, *, n, e):
    nodes = nodes_ref[...].reshape(n, H)
    ea = eatt_ref[...].reshape(1, e)
    src = src_ref[...].reshape(1, e)
    dst = dst_ref[...].reshape(1, e)
    q = q_ref[...].reshape(1, H)
    n_att = _dotT(nodes, wn_ref[...])
    q_att = _dotT(q, wg_ref[...])
    rows = jax.lax.broadcasted_iota(jnp.int32, (n, e), 0)
    oh_src = (rows == src).astype(f32)
    oh_dst = (rows == dst).astype(f32)
    gat = jnp.sum(oh_src * n_att, axis=0, keepdims=True)
    s = jnp.tanh(gat + ea + q_att)
    mx = jnp.max(jnp.where(oh_dst > 0.5, s, -1e30), axis=1, keepdims=True)
    m_e = jnp.sum(oh_dst * mx, axis=0, keepdims=True)
    ex = jnp.exp(s - m_e)
    den = jnp.sum(oh_dst * ex, axis=1, keepdims=True)
    den_e = jnp.sum(oh_dst * den, axis=0, keepdims=True)
    alpha = ex / (den_e + 1e-9)
    adj = _dotT(oh_dst * alpha, oh_src)
    aggn_ref[...] = _mm(adj, nodes)[None]


def _run_reason(nodes3, eatt, src, dst, q3, wn, wg, n, e):
    import functools
    spec = lambda shape: pl.BlockSpec((1,) + shape, lambda i: (i, 0, 0))
    full = lambda a: pl.BlockSpec(a.shape, lambda i: (0,) * a.ndim)
    return pl.pallas_call(
        functools.partial(_reason_body, n=n, e=e),
        grid=(B,),
        in_specs=[spec((n, H)), spec((1, e)), spec((1, e)), spec((1, e)),
                  spec((1, H)), full(wn), full(wg)],
        out_specs=spec((n, H)),
        out_shape=jax.ShapeDtypeStruct((B, n, H), f32),
    )(nodes3, eatt, src, dst, q3, wn, wg)


def _post_body(nodes_ref, aggn_ref, wm_ref, aw_ref, v_ref, q_ref,
               nn_ref, out_ref, *, n, gb):
    msg = _mm(aggn_ref[...], wm_ref[...])
    nn = jax.nn.relu(nodes_ref[...] + msg)
    nn_ref[...] = nn
    a1 = _mm(nn, aw_ref[:H, :])
    qw = _mm(q_ref[...], aw_ref[H:, :])
    t3 = jnp.tanh(a1.reshape(gb, n, H) + qw[:, None, :])
    sc = jnp.sum(t3 * v_ref[...][None], axis=-1, keepdims=True)
    mx = jnp.max(sc, axis=1, keepdims=True)
    ex = jnp.exp(sc - mx)
    a = ex / jnp.sum(ex, axis=1, keepdims=True)
    out_ref[...] = jnp.sum(nn.reshape(gb, n, H) * a, axis=1)


def _run_post(nodes, aggn, wm, aw, vrow, q, n):
    import functools
    gb = 8
    grid = B // gb
    rb = gb * n
    rows = lambda: pl.BlockSpec((rb, H), lambda i: (i, 0))
    full = lambda a: pl.BlockSpec(a.shape, lambda i: (0,) * a.ndim)
    return pl.pallas_call(
        functools.partial(_post_body, n=n, gb=gb),
        grid=(grid,),
        in_specs=[rows(), rows(), full(wm), full(aw), full(vrow),
                  pl.BlockSpec((gb, H), lambda i: (i, 0))],
        out_specs=(rows(), pl.BlockSpec((gb, H), lambda i: (i, 0))),
        out_shape=(jax.ShapeDtypeStruct((B * n, H), f32),
                   jax.ShapeDtypeStruct((B, H), f32)),
    )(nodes, aggn, wm, aw, vrow, q)


def _rel_body(qi_ref, qk_ref, io_ref, ko_ref, iw_ref, ib_ref, kw_ref, kb_ref,
              ir_ref, kr_ref):
    iw = iw_ref[...]; kw = kw_ref[...]
    ir_ref[...] = (_mm(qi_ref[...], iw[:H, :]) + _mm(ko_ref[...], iw[H:, :])
                   + ib_ref[...])
    kr_ref[...] = (_mm(qk_ref[...], kw[:H, :]) + _mm(io_ref[...], kw[H:, :])
                   + kb_ref[...])


def _run_rel(qi, qk, io1, ko1, iw, ib, kw, kb):
    return pl.pallas_call(
        _rel_body,
        out_shape=(jax.ShapeDtypeStruct((B, H), f32),
                   jax.ShapeDtypeStruct((B, H), f32)),
    )(qi, qk, io1, ko1, iw, ib, kw, kb)


def _head_body(io1_ref, io2_ref, ko1_ref, ko2_ref, gi_ref, gk_ref, pg_ref,
               hw_ref, out_ref):
    img_vec = io1_ref[...] + io2_ref[...]
    kg_vec = ko1_ref[...] + ko2_ref[...]
    gate = jax.nn.sigmoid(_mm(img_vec, gi_ref[...]) + _mm(kg_vec, gk_ref[...]))
    fused = gate * img_vec + (1.0 - gate) * kg_vec
    fused = fused * jax.nn.sigmoid(_mm(fused, pg_ref[...]))
    out_ref[...] = _mm(fused, hw_ref[...])


def _run_head(io1, io2, ko1, ko2, gi, gk, pg, hw):
    return pl.pallas_call(
        _head_body,
        out_shape=jax.ShapeDtypeStruct((B, NA), f32),
    )(io1, io2, ko1, ko2, gi, gk, pg, hw)


def kernel(question, question_mask, img_feat, img_loc, img_node1_id_list,
           img_node2_id_list, kg_entity, kg_edge, kg_node1_ids_list,
           kg_node2_ids_list, params):
    p = params
    row = lambda a: a.reshape(1, -1).astype(f32)

    emb = p['word_emb'][question]
    emb_t = jnp.swapaxes(emb, 0, 1).reshape(L * B, EMB)
    all_out, h_last = _run_lstm(emb_t, p['lstm_Wx'], p['lstm_Wh'],
                                row(p['lstm_b']))
    ao3 = all_out.reshape(L, B, H)
    mask_t = jnp.swapaxes(question_mask, 0, 1)[:, :, None]
    ques_img, ques_kg = _run_qpool(
        ao3, h_last, mask_t,
        p['qi_W'], row(p['qi_b']), p['qk_W'], row(p['qk_b']),
        row(p['qia_W'][:H, 0]), row(p['qia_W'][H:, 0]), row(p['qia_b']),
        row(p['qka_W'][:H, 0]), row(p['qka_W'][H:, 0]), row(p['qka_b']))

    loc_t = jnp.swapaxes(img_loc.reshape(B * EI, LOC), 0, 1)
    img_nodes, ie1, ie2 = _run_inode(
        img_feat.reshape(B * NO, IMG), p['inode_W'], row(p['inode_b']),
        loc_t, p['iedge_W'], row(p['iedge_b']),
        row(p['img_att_W'][H:2 * H, 0]), row(p['imgx_att_W'][H:2 * H, 0]))
    ie1 = ie1.reshape(B, 1, EI)
    ie2 = ie2.reshape(B, 1, EI)

    table_pad = jnp.pad(p['word_emb'], ((0, 0), (0, _DP - EMB)))
    ids_pad = jnp.concatenate([
        kg_entity.reshape(-1), kg_edge.reshape(-1),
        jnp.zeros((32 * _GPW - B * (NK + EK)) * LK, kg_entity.dtype)])
    sums = _run_sc_gather(table_pad, ids_pad.astype(jnp.int32))
    ent_sum = sums[:B * NK, :EMB].reshape(B, NK, EMB)
    edg_sum = sums[B * NK:B * (NK + EK), :EMB].reshape(B, EK, EMB)
    kg_nodes3, ke1, ke2 = _run_kgfeat(
        ent_sum, kg_entity.reshape(B, NK, LK).astype(f32),
        edg_sum, kg_edge.reshape(B, EK, LK).astype(f32),
        p['knode_W'], row(p['knode_b']), p['kedge_W'], row(p['kedge_b']),
        row(p['kg_att_W'][H:2 * H, 0]), row(p['kgx_att_W'][H:2 * H, 0]))
    ke1 = ke1.reshape(B, 1, EK)
    ke2 = ke2.reshape(B, 1, EK)

    n1 = img_node1_id_list[:, None, :]
    n2 = img_node2_id_list[:, None, :]
    k1 = kg_node1_ids_list[:, None, :]
    k2 = kg_node2_ids_list[:, None, :]

    img_nodes3 = img_nodes.reshape(B, NO, H)
    aggn_i1 = _run_reason(img_nodes3, ie1, n1, n2, ques_img[:, None, :],
                          row(p['img_att_W'][:H, 0]),
                          row(p['img_att_W'][2 * H:, 0]), NO, EI)
    img_nodes1, img_out_1 = _run_post(
        img_nodes, aggn_i1.reshape(B * NO, H), p['img_msg_W'],
        p['img_agg_W'], row(p['img_agg_v'][:, 0]), ques_img, NO)

    aggn_k1 = _run_reason(kg_nodes3, ke1, k1, k2, ques_kg[:, None, :],
                          row(p['kg_att_W'][:H, 0]),
                          row(p['kg_att_W'][2 * H:, 0]), NK, EK)
    kg_nodes1, kg_out_1 = _run_post(
        kg_nodes3.reshape(B * NK, H), aggn_k1.reshape(B * NK, H),
        p['kg_msg_W'], p['kg_agg_W'], row(p['kg_agg_v'][:, 0]), ques_kg, NK)

    img_rel, kg_rel = _run_rel(ques_img, ques_kg, img_out_1, kg_out_1,
                               p['img_rel_W'], row(p['img_rel_b']),
                               p['kg_rel_W'], row(p['kg_rel_b']))

    aggn_i2 = _run_reason(img_nodes1.reshape(B, NO, H), ie2, n1, n2,
                          img_rel[:, None, :],
                          row(p['imgx_att_W'][:H, 0]),
                          row(p['imgx_att_W'][2 * H:, 0]), NO, EI)
    _, img_out_2 = _run_post(
        img_nodes1, aggn_i2.reshape(B * NO, H), p['imgx_msg_W'],
        p['imgx_agg_W'], row(p['imgx_agg_v'][:, 0]), img_rel, NO)

    aggn_k2 = _run_reason(kg_nodes1.reshape(B, NK, H), ke2, k1, k2,
                          kg_rel[:, None, :],
                          row(p['kgx_att_W'][:H, 0]),
                          row(p['kgx_att_W'][2 * H:, 0]), NK, EK)
    _, kg_out_2 = _run_post(
        kg_nodes1, aggn_k2.reshape(B * NK, H), p['kgx_msg_W'],
        p['kgx_agg_W'], row(p['kgx_agg_v'][:, 0]), kg_rel, NK)

    return _run_head(img_out_1, img_out_2, kg_out_1, kg_out_2,
                     p['img_gate_W'], p['kg_gate_W'], p['pred_gate_W'],
                     p['head_W'])


# speedup vs baseline: 12.7253x; 1.4470x over previous
import functools

import jax
import jax.numpy as jnp
from jax.experimental import pallas as pl
from jax.experimental.pallas import tpu as pltpu
from jax.experimental.pallas import tpu_sc as plsc

B = 32; L = 20; H = 1024; EMB = 300; VOCAB = 10000; NA = 3000
IMG = 2048; LOC = 5; NO = 36; EI = 1260; NK = 100; EK = 200; LK = 10

f32 = jnp.float32


def _dotT(a, b):
    return jax.lax.dot_general(a, b, (((1,), (1,)), ((), ())),
                               preferred_element_type=f32)


def _mm(a, b):
    return jnp.dot(a, b, preferred_element_type=f32)


_DP = 384
_GPW = 304
_CG = 8
_NCHUNK = _GPW // _CG
_NLC = (EMB + 15) // 16


def _sc_gather_body(table_ref, ids_ref, out_ref,
                    idx0, idx1, rows0, rows1, out_v, sem0, sem1):
    wid = jax.lax.axis_index("s") * 2 + jax.lax.axis_index("c")

    def fire(c, idx_v, rows_v, sem):
        g0 = wid * _GPW + c * _CG
        pltpu.sync_copy(ids_ref.at[pl.ds(g0 * LK, _CG * LK)], idx_v)
        pltpu.async_copy(table_ref.at[idx_v], rows_v, sem)

    def compute(c, rows_v, sem):
        pltpu.make_async_copy(table_ref.at[idx0], rows_v, sem).wait()

        def group(g, carry2):
            for j in range(_NLC):
                sl = pl.ds(j * 16, 16)
                acc = rows_v[g * LK, sl]
                for r in range(1, LK):
                    acc = acc + rows_v[g * LK + r, sl]
                out_v[g, sl] = acc
            return carry2

        jax.lax.fori_loop(0, _CG, group, 0)
        g0 = wid * _GPW + c * _CG
        pltpu.sync_copy(out_v, out_ref.at[pl.ds(g0, _CG)])

    fire(0, idx0, rows0, sem0)

    def pair(i, carry):
        c = i * 2
        fire(c + 1, idx1, rows1, sem1)
        compute(c, rows0, sem0)

        @pl.when(c + 2 < _NCHUNK)
        def _():
            fire(c + 2, idx0, rows0, sem0)

        compute(c + 1, rows1, sem1)
        return carry

    jax.lax.fori_loop(0, _NCHUNK // 2, pair, 0)


def _run_sc_gather(table_pad, ids_pad):
    k = pl.kernel(
        _sc_gather_body,
        mesh=plsc.VectorSubcoreMesh(core_axis_name="c", subcore_axis_name="s"),
        out_type=jax.ShapeDtypeStruct((32 * _GPW, _DP), f32),
        scratch_types=[
            pltpu.VMEM((_CG * LK,), jnp.int32),
            pltpu.VMEM((_CG * LK,), jnp.int32),
            pltpu.VMEM((_CG * LK, _DP), f32),
            pltpu.VMEM((_CG * LK, _DP), f32),
            pltpu.VMEM((_CG, _DP), f32),
            pltpu.SemaphoreType.DMA,
            pltpu.SemaphoreType.DMA,
        ],
    )
    return k(table_pad, ids_pad)


def _lstm_body(emb_ref, wx_ref, wh_ref, b_ref, allout_ref, hlast_ref, x_scr):
    x_scr[...] = _mm(emb_ref[...], wx_ref[...]) + b_ref[...]

    def step(t, carry):
        h, c = carry
        z = x_scr[pl.ds(t * B, B), :] + _mm(h, wh_ref[...])
        i = z[:, :H]; f = z[:, H:2 * H]; g = z[:, 2 * H:3 * H]; o = z[:, 3 * H:]
        c = jax.nn.sigmoid(f) * c + jax.nn.sigmoid(i) * jnp.tanh(g)
        h = jax.nn.sigmoid(o) * jnp.tanh(c)
        allout_ref[pl.ds(t * B, B), :] = h
        return (h, c)

    z0 = jnp.zeros((B, H), f32)
    h, _ = jax.lax.fori_loop(0, L, step, (z0, z0))
    hlast_ref[...] = h


def _run_lstm(emb_t, wx, wh, b):
    return pl.pallas_call(
        _lstm_body,
        out_shape=(jax.ShapeDtypeStruct((L * B, H), f32),
                   jax.ShapeDtypeStruct((B, H), f32)),
        scratch_shapes=[pltpu.VMEM((L * B, 4 * H), f32)],
    )(emb_t, wx, wh, b)


def _qpool_body(ao_ref, h_ref, m_ref, qiw_ref, qib_ref, qkw_ref, qkb_ref,
                w1i_ref, w2i_ref, bi_ref, w1k_ref, w2k_ref, bk_ref,
                qimg_ref, qkg_ref):
    ao = ao_ref[...]
    h = h_ref[...]
    mask = m_ref[...]

    def pool(qw, qb, w1, w2, ab):
        q = _mm(h, qw) + qb
        s = (jnp.sum(ao * w1[None], axis=-1, keepdims=True)
             + jnp.sum(q * w2, axis=-1, keepdims=True)[None]
             + ab[None])
        s = jnp.where(mask == 1, -1e32, s)
        mx = jnp.max(s, axis=0, keepdims=True)
        ex = jnp.exp(s - mx)
        a = ex / jnp.sum(ex, axis=0, keepdims=True)
        return jnp.sum(ao * a, axis=0)

    qimg_ref[...] = pool(qiw_ref[...], qib_ref[...], w1i_ref[...],
                         w2i_ref[...], bi_ref[...])
    qkg_ref[...] = pool(qkw_ref[...], qkb_ref[...], w1k_ref[...],
                        w2k_ref[...], bk_ref[...])


def _run_qpool(ao3, h, mask_t, qiw, qib, qkw, qkb, w1i, w2i, bi, w1k, w2k, bk):
    return pl.pallas_call(
        _qpool_body,
        out_shape=(jax.ShapeDtypeStruct((B, H), f32),
                   jax.ShapeDtypeStruct((B, H), f32)),
    )(ao3, h, mask_t, qiw, qib, qkw, qkb, w1i, w2i, bi, w1k, w2k, bk)


def _inode_body(feat_ref, w_ref, b_ref, loc_ref, ew_ref, eb_ref,
                wa1_ref, wa2_ref, nodes_ref, e1_ref, e2_ref):
    nodes_ref[...] = _mm(feat_ref[...], w_ref[...]) + b_ref[...]
    loc = loc_ref[...]

    def eatt(wa):
        fold = _dotT(ew_ref[...], wa)
        c = _dotT(eb_ref[...], wa)
        return jnp.sum(loc * fold, axis=0, keepdims=True) + c

    e1_ref[...] = eatt(wa1_ref[...])
    e2_ref[...] = eatt(wa2_ref[...])


def _run_inode(feat, w, b, loc_t, ew, eb, wa1, wa2):
    return pl.pallas_call(
        _inode_body,
        out_shape=(jax.ShapeDtypeStruct((B * NO, H), f32),
                   jax.ShapeDtypeStruct((1, B * EI), f32),
                   jax.ShapeDtypeStruct((1, B * EI), f32)),
    )(feat, w, b, loc_t, ew, eb, wa1, wa2)


def _kgfeat_body(ent_ref, eid_ref, edg_ref, gid_ref, nw_ref, nb_ref,
                 ew_ref, eb_ref, wa1_ref, wa2_ref,
                 nodes_ref, e1_ref, e2_ref):
    ent = ent_ref[...].reshape(NK, EMB)
    eid = eid_ref[...].reshape(NK, LK)
    elen = jnp.maximum(jnp.sum((eid != 1.0).astype(f32), axis=-1,
                               keepdims=True), 1.0)
    efeat = ent / elen
    nodes_ref[...] = (_mm(efeat, nw_ref[...]) + nb_ref[...])[None]

    edg = edg_ref[...].reshape(EK, EMB)
    gid = gid_ref[...].reshape(EK, LK)
    glen = jnp.maximum(jnp.sum((gid != 1.0).astype(f32), axis=-1,
                               keepdims=True), 1.0)
    gfeat = edg / glen

    def eatt(wa):
        fold = _dotT(ew_ref[...], wa)
        c = _dotT(eb_ref[...], wa)
        return _mm(gfeat, fold) + c

    e1_ref[...] = eatt(wa1_ref[...])[None]
    e2_ref[...] = eatt(wa2_ref[...])[None]


def _run_kgfeat(ent, eid, edg, gid, nw, nb, ew, eb, wa1, wa2):
    spec = lambda shape: pl.BlockSpec((1,) + shape, lambda i: (i, 0, 0))
    full = lambda a: pl.BlockSpec(a.shape, lambda i: (0,) * a.ndim)
    return pl.pallas_call(
        _kgfeat_body,
        grid=(B,),
        in_specs=[spec((NK, EMB)), spec((NK, LK)),
                  spec((EK, EMB)), spec((EK, LK)),
                  full(nw), full(nb), full(ew), full(eb), full(wa1), full(wa2)],
        out_specs=(spec((NK, H)), spec((EK, 1)), spec((EK, 1))),
        out_shape=(jax.ShapeDtypeStruct((B, NK, H), f32),
                   jax.ShapeDtypeStruct((B, EK, 1), f32),
                   jax.ShapeDtypeStruct((B, EK, 1), f32)),
    )(ent, eid, edg, gid, nw, nb, ew, eb, wa1, wa2)


def _round_body(nodes_ref, eatt_ref, src_ref, dst_ref, q_ref, wn_ref, wg_ref,
                wm_ref, aw_ref, v_ref, nn_ref, out_ref, *, n, e, gb):
    nodes2 = nodes_ref[...]
    nodes3 = nodes2.reshape(gb, n, H)
    ea = eatt_ref[...]
    src = src_ref[...]
    dst = dst_ref[...]
    q = q_ref[...]
    wn = wn_ref[...]
    wg = wg_ref[...]
    n_att = jnp.sum(nodes3 * wn[None], axis=-1, keepdims=True)
    q_att = jnp.sum(q * wg, axis=-1, keepdims=True)[:, :, None]
    rows = jax.lax.broadcasted_iota(jnp.int32, (gb, n, e), 1)
    oh_src = (rows == src).astype(f32)
    oh_dst = (rows == dst).astype(f32)
    gat = jnp.sum(oh_src * n_att, axis=1, keepdims=True)
    s = jnp.tanh(gat + ea + q_att)
    mx = jnp.max(jnp.where(oh_dst > 0.5, s, -1e30), axis=2, keepdims=True)
    m_e = jnp.sum(oh_dst * mx, axis=1, keepdims=True)
    ex = jnp.exp(s - m_e)
    den = jnp.sum(oh_dst * ex, axis=2, keepdims=True)
    den_e = jnp.sum(oh_dst * den, axis=1, keepdims=True)
    alpha = ex / (den_e + 1e-9)
    adj = jax.lax.dot_general(oh_dst * alpha, oh_src,
                              (((2,), (2,)), ((0,), (0,))),
                              preferred_element_type=f32)
    aggn = jax.lax.dot_general(adj, nodes3, (((2,), (1,)), ((0,), (0,))),
                               preferred_element_type=f32)
    msg = _mm(aggn.reshape(gb * n, H), wm_ref[...])
    nn = jax.nn.relu(nodes2 + msg)
    nn_ref[...] = nn
    a1 = _mm(nn, aw_ref[:H, :])
    qw = _mm(q, aw_ref[H:, :])
    t3 = jnp.tanh(a1.reshape(gb, n, H) + qw[:, None, :])
    sc = jnp.sum(t3 * v_ref[...][None], axis=-1, keepdims=True)
    mx2 = jnp.max(sc, axis=1, keepdims=True)
    ex2 = jnp.exp(sc - mx2)
    a = ex2 / jnp.sum(ex2, axis=1, keepdims=True)
    out_ref[...] = jnp.sum(nn.reshape(gb, n, H) * a, axis=1)


def _run_round(nodes, eatt, src, dst, q, wn, wg, wm, aw, vrow, n, e):
    gb = 8
    rows = pl.BlockSpec((gb * n, H), lambda i: (i, 0))
    edges = pl.BlockSpec((gb, 1, e), lambda i: (i, 0, 0))
    qspec = pl.BlockSpec((gb, H), lambda i: (i, 0))
    full = lambda a: pl.BlockSpec(a.shape, lambda i: (0,) * a.ndim)
    return pl.pallas_call(
        functools.partial(_round_body, n=n, e=e, gb=gb),
        grid=(B // gb,),
        in_specs=[rows, edges, edges, edges, qspec,
                  full(wn), full(wg), full(wm), full(aw), full(vrow)],
        out_specs=(rows, qspec),
        out_shape=(jax.ShapeDtypeStruct((B * n, H), f32),
                   jax.ShapeDtypeStruct((B, H), f32)),
    )(nodes, eatt, src, dst, q, wn, wg, wm, aw, vrow)


def _rel_body(qi_ref, qk_ref, io_ref, ko_ref, iw_ref, ib_ref, kw_ref, kb_ref,
              ir_ref, kr_ref):
    iw = iw_ref[...]; kw = kw_ref[...]
    ir_ref[...] = (_mm(qi_ref[...], iw[:H, :]) + _mm(ko_ref[...], iw[H:, :])
                   + ib_ref[...])
    kr_ref[...] = (_mm(qk_ref[...], kw[:H, :]) + _mm(io_ref[...], kw[H:, :])
                   + kb_ref[...])


def _run_rel(qi, qk, io1, ko1, iw, ib, kw, kb):
    return pl.pallas_call(
        _rel_body,
        out_shape=(jax.ShapeDtypeStruct((B, H), f32),
                   jax.ShapeDtypeStruct((B, H), f32)),
    )(qi, qk, io1, ko1, iw, ib, kw, kb)


def _head_body(io1_ref, io2_ref, ko1_ref, ko2_ref, gi_ref, gk_ref, pg_ref,
               hw_ref, out_ref):
    img_vec = io1_ref[...] + io2_ref[...]
    kg_vec = ko1_ref[...] + ko2_ref[...]
    gate = jax.nn.sigmoid(_mm(img_vec, gi_ref[...]) + _mm(kg_vec, gk_ref[...]))
    fused = gate * img_vec + (1.0 - gate) * kg_vec
    fused = fused * jax.nn.sigmoid(_mm(fused, pg_ref[...]))
    out_ref[...] = _mm(fused, hw_ref[...])


def _run_head(io1, io2, ko1, ko2, gi, gk, pg, hw):
    return pl.pallas_call(
        _head_body,
        out_shape=jax.ShapeDtypeStruct((B, NA), f32),
    )(io1, io2, ko1, ko2, gi, gk, pg, hw)


def kernel(question, question_mask, img_feat, img_loc, img_node1_id_list,
           img_node2_id_list, kg_entity, kg_edge, kg_node1_ids_list,
           kg_node2_ids_list, params):
    p = params
    row = lambda a: a.reshape(1, -1).astype(f32)

    emb = p['word_emb'][question]
    emb_t = jnp.swapaxes(emb, 0, 1).reshape(L * B, EMB)
    all_out, h_last = _run_lstm(emb_t, p['lstm_Wx'], p['lstm_Wh'],
                                row(p['lstm_b']))
    ao3 = all_out.reshape(L, B, H)
    mask_t = jnp.swapaxes(question_mask, 0, 1)[:, :, None]
    ques_img, ques_kg = _run_qpool(
        ao3, h_last, mask_t,
        p['qi_W'], row(p['qi_b']), p['qk_W'], row(p['qk_b']),
        row(p['qia_W'][:H, 0]), row(p['qia_W'][H:, 0]), row(p['qia_b']),
        row(p['qka_W'][:H, 0]), row(p['qka_W'][H:, 0]), row(p['qka_b']))

    loc_t = jnp.swapaxes(img_loc.reshape(B * EI, LOC), 0, 1)
    img_nodes, ie1, ie2 = _run_inode(
        img_feat.reshape(B * NO, IMG), p['inode_W'], row(p['inode_b']),
        loc_t, p['iedge_W'], row(p['iedge_b']),
        row(p['img_att_W'][H:2 * H, 0]), row(p['imgx_att_W'][H:2 * H, 0]))
    ie1 = ie1.reshape(B, 1, EI)
    ie2 = ie2.reshape(B, 1, EI)

    table_pad = jnp.pad(p['word_emb'], ((0, 0), (0, _DP - EMB)))
    ids_pad = jnp.concatenate([
        kg_entity.reshape(-1), kg_edge.reshape(-1),
        jnp.zeros((32 * _GPW - B * (NK + EK)) * LK, kg_entity.dtype)])
    sums = _run_sc_gather(table_pad, ids_pad.astype(jnp.int32))
    ent_sum = sums[:B * NK, :EMB].reshape(B, NK, EMB)
    edg_sum = sums[B * NK:B * (NK + EK), :EMB].reshape(B, EK, EMB)
    kg_nodes3, ke1, ke2 = _run_kgfeat(
        ent_sum, kg_entity.reshape(B, NK, LK).astype(f32),
        edg_sum, kg_edge.reshape(B, EK, LK).astype(f32),
        p['knode_W'], row(p['knode_b']), p['kedge_W'], row(p['kedge_b']),
        row(p['kg_att_W'][H:2 * H, 0]), row(p['kgx_att_W'][H:2 * H, 0]))
    ke1 = ke1.reshape(B, 1, EK)
    ke2 = ke2.reshape(B, 1, EK)

    n1 = img_node1_id_list[:, None, :]
    n2 = img_node2_id_list[:, None, :]
    k1 = kg_node1_ids_list[:, None, :]
    k2 = kg_node2_ids_list[:, None, :]

    img_nodes1, img_out_1 = _run_round(
        img_nodes, ie1, n1, n2, ques_img,
        row(p['img_att_W'][:H, 0]), row(p['img_att_W'][2 * H:, 0]),
        p['img_msg_W'], p['img_agg_W'], row(p['img_agg_v'][:, 0]), NO, EI)

    kg_nodes1, kg_out_1 = _run_round(
        kg_nodes3.reshape(B * NK, H), ke1, k1, k2, ques_kg,
        row(p['kg_att_W'][:H, 0]), row(p['kg_att_W'][2 * H:, 0]),
        p['kg_msg_W'], p['kg_agg_W'], row(p['kg_agg_v'][:, 0]), NK, EK)

    img_rel, kg_rel = _run_rel(ques_img, ques_kg, img_out_1, kg_out_1,
                               p['img_rel_W'], row(p['img_rel_b']),
                               p['kg_rel_W'], row(p['kg_rel_b']))

    _, img_out_2 = _run_round(
        img_nodes1, ie2, n1, n2, img_rel,
        row(p['imgx_att_W'][:H, 0]), row(p['imgx_att_W'][2 * H:, 0]),
        p['imgx_msg_W'], p['imgx_agg_W'], row(p['imgx_agg_v'][:, 0]), NO, EI)

    _, kg_out_2 = _run_round(
        kg_nodes1, ke2, k1, k2, kg_rel,
        row(p['kgx_att_W'][:H, 0]), row(p['kgx_att_W'][2 * H:, 0]),
        p['kgx_msg_W'], p['kgx_agg_W'], row(p['kgx_agg_v'][:, 0]), NK, EK)

    return _run_head(img_out_1, img_out_2, kg_out_1, kg_out_2,
                     p['img_gate_W'], p['kg_gate_W'], p['pred_gate_W'],
                     p['head_W'])


# speedup vs baseline: 14.8532x; 1.1672x over previous
import functools

import jax
import jax.numpy as jnp
from jax.experimental import pallas as pl
from jax.experimental.pallas import tpu as pltpu
from jax.experimental.pallas import tpu_sc as plsc

B = 32; L = 20; H = 1024; EMB = 300; VOCAB = 10000; NA = 3000
IMG = 2048; LOC = 5; NO = 36; EI = 1260; NK = 100; EK = 200; LK = 10

f32 = jnp.float32


def _dotT(a, b):
    return jax.lax.dot_general(a, b, (((1,), (1,)), ((), ())),
                               preferred_element_type=f32)


def _mm(a, b):
    return jnp.dot(a, b, preferred_element_type=f32)


_DP = 384
_GPW = 304
_CG = 8
_NCHUNK = _GPW // _CG
_NLC = (EMB + 15) // 16


def _sc_gather_body(table_ref, ids_ref, out_ref,
                    ids_all, rows0, rows1, out_v, sem0, sem1):
    wid = jax.lax.axis_index("s") * 2 + jax.lax.axis_index("c")
    pltpu.sync_copy(ids_ref.at[pl.ds(wid * _GPW * LK, _GPW * LK)], ids_all)

    def fire(c, rows_v, sem):
        pltpu.async_copy(
            table_ref.at[ids_all.at[pl.ds(c * _CG * LK, _CG * LK)]],
            rows_v, sem)

    def compute(c, rows_v, sem):
        pltpu.make_async_copy(table_ref.at[ids_all.at[:_CG * LK]],
                              rows_v, sem).wait()

        def group(g, carry2):
            for j in range(_NLC):
                sl = pl.ds(j * 16, 16)
                acc = rows_v[g * LK, sl]
                for r in range(1, LK):
                    acc = acc + rows_v[g * LK + r, sl]
                out_v[g, sl] = acc
            for j in range(_NLC, _DP // 16):
                out_v[g, pl.ds(j * 16, 16)] = jnp.zeros((16,), f32)
            return carry2

        jax.lax.fori_loop(0, _CG, group, 0)
        g0 = wid * _GPW + c * _CG
        pltpu.sync_copy(out_v, out_ref.at[pl.ds(g0, _CG)])

    fire(0, rows0, sem0)

    def pair(i, carry):
        c = i * 2
        fire(c + 1, rows1, sem1)
        compute(c, rows0, sem0)

        @pl.when(c + 2 < _NCHUNK)
        def _():
            fire(c + 2, rows0, sem0)

        compute(c + 1, rows1, sem1)
        return carry

    jax.lax.fori_loop(0, _NCHUNK // 2, pair, 0)


def _run_sc_gather(table_pad, ids_pad):
    k = pl.kernel(
        _sc_gather_body,
        mesh=plsc.VectorSubcoreMesh(core_axis_name="c", subcore_axis_name="s"),
        out_type=jax.ShapeDtypeStruct((32 * _GPW, _DP), f32),
        scratch_types=[
            pltpu.VMEM((_GPW * LK,), jnp.int32),
            pltpu.VMEM((_CG * LK, _DP), f32),
            pltpu.VMEM((_CG * LK, _DP), f32),
            pltpu.VMEM((_CG, _DP), f32),
            pltpu.SemaphoreType.DMA,
            pltpu.SemaphoreType.DMA,
        ],
    )
    return k(table_pad, ids_pad)


def _tpad_body(x_ref, o_ref):
    o_ref[:, :EMB] = x_ref[...]
    o_ref[:, EMB:] = jnp.zeros((x_ref.shape[0], _DP - EMB), f32)


def _run_table_pad(table):
    blk = 2000
    return pl.pallas_call(
        _tpad_body,
        grid=(VOCAB // blk,),
        in_specs=[pl.BlockSpec((blk, EMB), lambda i: (i, 0))],
        out_specs=pl.BlockSpec((blk, _DP), lambda i: (i, 0)),
        out_shape=jax.ShapeDtypeStruct((VOCAB, _DP), f32),
    )(table)


def _lstm_body(emb_ref, wx_ref, wh_ref, b_ref, allout_ref, hlast_ref, x_scr):
    x_scr[...] = _mm(emb_ref[...], wx_ref[...]) + b_ref[...]

    def step(t, carry):
        h, c = carry
        z = x_scr[pl.ds(t * B, B), :] + _mm(h, wh_ref[...])
        i = z[:, :H]; f = z[:, H:2 * H]; g = z[:, 2 * H:3 * H]; o = z[:, 3 * H:]
        c = jax.nn.sigmoid(f) * c + jax.nn.sigmoid(i) * jnp.tanh(g)
        h = jax.nn.sigmoid(o) * jnp.tanh(c)
        allout_ref[pl.ds(t * B, B), :] = h
        return (h, c)

    z0 = jnp.zeros((B, H), f32)
    h, _ = jax.lax.fori_loop(0, L, step, (z0, z0))
    hlast_ref[...] = h


def _run_lstm(emb_t, wx, wh, b):
    return pl.pallas_call(
        _lstm_body,
        out_shape=(jax.ShapeDtypeStruct((L * B, H), f32),
                   jax.ShapeDtypeStruct((B, H), f32)),
        scratch_shapes=[pltpu.VMEM((L * B, 4 * H), f32)],
    )(emb_t, wx, wh, b)


def _qpool_body(ao_ref, h_ref, m_ref, qiw_ref, qib_ref, qkw_ref, qkb_ref,
                w1i_ref, w2i_ref, bi_ref, w1k_ref, w2k_ref, bk_ref,
                qimg_ref, qkg_ref):
    ao = ao_ref[...]
    h = h_ref[...]
    mask = m_ref[...]

    def pool(qw, qb, w1, w2, ab):
        q = _mm(h, qw) + qb
        s = (jnp.sum(ao * w1[None], axis=-1, keepdims=True)
             + jnp.sum(q * w2, axis=-1, keepdims=True)[None]
             + ab[None])
        s = jnp.where(mask == 1, -1e32, s)
        mx = jnp.max(s, axis=0, keepdims=True)
        ex = jnp.exp(s - mx)
        a = ex / jnp.sum(ex, axis=0, keepdims=True)
        return jnp.sum(ao * a, axis=0)

    qimg_ref[...] = pool(qiw_ref[...], qib_ref[...], w1i_ref[...],
                         w2i_ref[...], bi_ref[...])
    qkg_ref[...] = pool(qkw_ref[...], qkb_ref[...], w1k_ref[...],
                        w2k_ref[...], bk_ref[...])


def _run_qpool(ao3, h, mask_t, qiw, qib, qkw, qkb, w1i, w2i, bi, w1k, w2k, bk):
    return pl.pallas_call(
        _qpool_body,
        out_shape=(jax.ShapeDtypeStruct((B, H), f32),
                   jax.ShapeDtypeStruct((B, H), f32)),
    )(ao3, h, mask_t, qiw, qib, qkw, qkb, w1i, w2i, bi, w1k, w2k, bk)


def _inode_body(feat_ref, w_ref, b_ref, loc_ref, ew_ref, eb_ref,
                wa1_ref, wa2_ref, nodes_ref, e1_ref, e2_ref):
    nodes_ref[...] = _mm(feat_ref[...], w_ref[...]) + b_ref[...]
    loc = loc_ref[...]

    def eatt(wa):
        fold = _dotT(ew_ref[...], wa)
        c = _dotT(eb_ref[...], wa)
        return jnp.sum(loc * fold, axis=0, keepdims=True) + c

    e1_ref[...] = eatt(wa1_ref[...])
    e2_ref[...] = eatt(wa2_ref[...])


def _run_inode(feat, w, b, loc_t, ew, eb, wa1, wa2):
    return pl.pallas_call(
        _inode_body,
        out_shape=(jax.ShapeDtypeStruct((B * NO, H), f32),
                   jax.ShapeDtypeStruct((1, B * EI), f32),
                   jax.ShapeDtypeStruct((1, B * EI), f32)),
    )(feat, w, b, loc_t, ew, eb, wa1, wa2)


_KGB = 8


def _kgfeat_body(ent_ref, edg_ref, eid_ref, gid_ref, nw_ref, nb_ref,
                 ew_ref, eb_ref, wa1_ref, wa2_ref,
                 nodes_ref, e1_ref, e2_ref):
    eid = eid_ref[...].reshape(_KGB * NK, LK)
    elen = jnp.maximum(jnp.sum((eid != 1.0).astype(f32), axis=-1,
                               keepdims=True), 1.0)
    efeat = ent_ref[...] / elen
    nodes_ref[...] = (_mm(efeat, nw_ref[...])
                      + nb_ref[...]).reshape(_KGB, NK, H)

    gid = gid_ref[...].reshape(_KGB * EK, LK)
    glen = jnp.maximum(jnp.sum((gid != 1.0).astype(f32), axis=-1,
                               keepdims=True), 1.0)
    gfeat = edg_ref[...] / glen

    def eatt(wa):
        fold = _dotT(ew_ref[...], wa)
        c = _dotT(eb_ref[...], wa)
        return _mm(gfeat, fold) + c

    e1_ref[...] = eatt(wa1_ref[...]).reshape(_KGB, EK, 1)
    e2_ref[...] = eatt(wa2_ref[...]).reshape(_KGB, EK, 1)


def _run_kgfeat(sums, eid, gid, nw, nb, ew, eb, wa1, wa2):
    spec = lambda shape: pl.BlockSpec((_KGB,) + shape, lambda i: (i, 0, 0))
    full = lambda a: pl.BlockSpec(a.shape, lambda i: (0,) * a.ndim)
    ent_spec = pl.BlockSpec((_KGB * NK, _DP), lambda i: (i, 0))
    edg_spec = pl.BlockSpec((_KGB * EK, _DP),
                            lambda i: (i + B * NK // (_KGB * EK), 0))
    return pl.pallas_call(
        _kgfeat_body,
        grid=(B // _KGB,),
        in_specs=[ent_spec, edg_spec, spec((NK, LK)), spec((EK, LK)),
                  full(nw), full(nb), full(ew), full(eb), full(wa1), full(wa2)],
        out_specs=(spec((NK, H)), spec((EK, 1)), spec((EK, 1))),
        out_shape=(jax.ShapeDtypeStruct((B, NK, H), f32),
                   jax.ShapeDtypeStruct((B, EK, 1), f32),
                   jax.ShapeDtypeStruct((B, EK, 1), f32)),
    )(sums, sums, eid, gid, nw, nb, ew, eb, wa1, wa2)


def _round_body(nodes_ref, eatt_ref, src_ref, dst_ref, q_ref, wn_ref, wg_ref,
                wm_ref, aw_ref, v_ref, nn_ref, out_ref, *, n, e, gb):
    nodes2 = nodes_ref[...]
    nodes3 = nodes2.reshape(gb, n, H)
    ea = eatt_ref[...]
    src = src_ref[...]
    dst = dst_ref[...]
    q = q_ref[...]
    wn = wn_ref[...]
    wg = wg_ref[...]
    n_att = jnp.sum(nodes3 * wn[None], axis=-1, keepdims=True)
    q_att = jnp.sum(q * wg, axis=-1, keepdims=True)[:, :, None]
    rows = jax.lax.broadcasted_iota(jnp.int32, (gb, n, e), 1)
    oh_src = (rows == src).astype(f32)
    oh_dst = (rows == dst).astype(f32)
    gat = jnp.sum(oh_src * n_att, axis=1, keepdims=True)
    s = jnp.tanh(gat + ea + q_att)
    mx = jnp.max(jnp.where(oh_dst > 0.5, s, -1e30), axis=2, keepdims=True)
    m_e = jnp.sum(oh_dst * mx, axis=1, keepdims=True)
    ex = jnp.exp(s - m_e)
    den = jnp.sum(oh_dst * ex, axis=2, keepdims=True)
    den_e = jnp.sum(oh_dst * den, axis=1, keepdims=True)
    alpha = ex / (den_e + 1e-9)
    adj = jax.lax.dot_general(oh_dst * alpha, oh_src,
                              (((2,), (2,)), ((0,), (0,))),
                              preferred_element_type=f32)
    aggn = jax.lax.dot_general(adj, nodes3, (((2,), (1,)), ((0,), (0,))),
                               preferred_element_type=f32)
    msg = _mm(aggn.reshape(gb * n, H), wm_ref[...])
    nn = jax.nn.relu(nodes2 + msg)
    nn_ref[...] = nn
    a1 = _mm(nn, aw_ref[:H, :])
    qw = _mm(q, aw_ref[H:, :])
    t3 = jnp.tanh(a1.reshape(gb, n, H) + qw[:, None, :])
    sc = jnp.sum(t3 * v_ref[...][None], axis=-1, keepdims=True)
    mx2 = jnp.max(sc, axis=1, keepdims=True)
    ex2 = jnp.exp(sc - mx2)
    a = ex2 / jnp.sum(ex2, axis=1, keepdims=True)
    out_ref[...] = jnp.sum(nn.reshape(gb, n, H) * a, axis=1)


def _run_round(nodes, eatt, src, dst, q, wn, wg, wm, aw, vrow, n, e):
    gb = 8
    rows = pl.BlockSpec((gb * n, H), lambda i: (i, 0))
    edges = pl.BlockSpec((gb, 1, e), lambda i: (i, 0, 0))
    qspec = pl.BlockSpec((gb, H), lambda i: (i, 0))
    full = lambda a: pl.BlockSpec(a.shape, lambda i: (0,) * a.ndim)
    return pl.pallas_call(
        functools.partial(_round_body, n=n, e=e, gb=gb),
        grid=(B // gb,),
        in_specs=[rows, edges, edges, edges, qspec,
                  full(wn), full(wg), full(wm), full(aw), full(vrow)],
        out_specs=(rows, qspec),
        out_shape=(jax.ShapeDtypeStruct((B * n, H), f32),
                   jax.ShapeDtypeStruct((B, H), f32)),
    )(nodes, eatt, src, dst, q, wn, wg, wm, aw, vrow)


def _rel_body(qi_ref, qk_ref, io_ref, ko_ref, iw_ref, ib_ref, kw_ref, kb_ref,
              ir_ref, kr_ref):
    iw = iw_ref[...]; kw = kw_ref[...]
    ir_ref[...] = (_mm(qi_ref[...], iw[:H, :]) + _mm(ko_ref[...], iw[H:, :])
                   + ib_ref[...])
    kr_ref[...] = (_mm(qk_ref[...], kw[:H, :]) + _mm(io_ref[...], kw[H:, :])
                   + kb_ref[...])


def _run_rel(qi, qk, io1, ko1, iw, ib, kw, kb):
    return pl.pallas_call(
        _rel_body,
        out_shape=(jax.ShapeDtypeStruct((B, H), f32),
                   jax.ShapeDtypeStruct((B, H), f32)),
    )(qi, qk, io1, ko1, iw, ib, kw, kb)


def _head_body(io1_ref, io2_ref, ko1_ref, ko2_ref, gi_ref, gk_ref, pg_ref,
               hw_ref, out_ref):
    img_vec = io1_ref[...] + io2_ref[...]
    kg_vec = ko1_ref[...] + ko2_ref[...]
    gate = jax.nn.sigmoid(_mm(img_vec, gi_ref[...]) + _mm(kg_vec, gk_ref[...]))
    fused = gate * img_vec + (1.0 - gate) * kg_vec
    fused = fused * jax.nn.sigmoid(_mm(fused, pg_ref[...]))
    out_ref[...] = _mm(fused, hw_ref[...])


def _run_head(io1, io2, ko1, ko2, gi, gk, pg, hw):
    return pl.pallas_call(
        _head_body,
        out_shape=jax.ShapeDtypeStruct((B, NA), f32),
    )(io1, io2, ko1, ko2, gi, gk, pg, hw)


def kernel(question, question_mask, img_feat, img_loc, img_node1_id_list,
           img_node2_id_list, kg_entity, kg_edge, kg_node1_ids_list,
           kg_node2_ids_list, params):
    p = params
    row = lambda a: a.reshape(1, -1).astype(f32)

    emb = p['word_emb'][question]
    emb_t = jnp.swapaxes(emb, 0, 1).reshape(L * B, EMB)
    all_out, h_last = _run_lstm(emb_t, p['lstm_Wx'], p['lstm_Wh'],
                                row(p['lstm_b']))
    ao3 = all_out.reshape(L, B, H)
    mask_t = jnp.swapaxes(question_mask, 0, 1)[:, :, None]
    ques_img, ques_kg = _run_qpool(
        ao3, h_last, mask_t,
        p['qi_W'], row(p['qi_b']), p['qk_W'], row(p['qk_b']),
        row(p['qia_W'][:H, 0]), row(p['qia_W'][H:, 0]), row(p['qia_b']),
        row(p['qka_W'][:H, 0]), row(p['qka_W'][H:, 0]), row(p['qka_b']))

    loc_t = jnp.swapaxes(img_loc.reshape(B * EI, LOC), 0, 1)
    img_nodes, ie1, ie2 = _run_inode(
        img_feat.reshape(B * NO, IMG), p['inode_W'], row(p['inode_b']),
        loc_t, p['iedge_W'], row(p['iedge_b']),
        row(p['img_att_W'][H:2 * H, 0]), row(p['imgx_att_W'][H:2 * H, 0]))
    ie1 = ie1.reshape(B, 1, EI)
    ie2 = ie2.reshape(B, 1, EI)

    table_pad = _run_table_pad(p['word_emb'])
    ids_pad = jnp.concatenate([
        kg_entity.reshape(-1), kg_edge.reshape(-1),
        jnp.zeros((32 * _GPW - B * (NK + EK)) * LK, kg_entity.dtype)])
    sums = _run_sc_gather(table_pad, ids_pad.astype(jnp.int32))
    nwp = jnp.pad(p['knode_W'], ((0, _DP - EMB), (0, 0)))
    ewp = jnp.pad(p['kedge_W'], ((0, _DP - EMB), (0, 0)))
    kg_nodes3, ke1, ke2 = _run_kgfeat(
        sums, kg_entity.reshape(B, NK, LK).astype(f32),
        kg_edge.reshape(B, EK, LK).astype(f32),
        nwp, row(p['knode_b']), ewp, row(p['kedge_b']),
        row(p['kg_att_W'][H:2 * H, 0]), row(p['kgx_att_W'][H:2 * H, 0]))
    ke1 = ke1.reshape(B, 1, EK)
    ke2 = ke2.reshape(B, 1, EK)

    n1 = img_node1_id_list[:, None, :]
    n2 = img_node2_id_list[:, None, :]
    k1 = kg_node1_ids_list[:, None, :]
    k2 = kg_node2_ids_list[:, None, :]

    img_nodes1, img_out_1 = _run_round(
        img_nodes, ie1, n1, n2, ques_img,
        row(p['img_att_W'][:H, 0]), row(p['img_att_W'][2 * H:, 0]),
        p['img_msg_W'], p['img_agg_W'], row(p['img_agg_v'][:, 0]), NO, EI)

    kg_nodes1, kg_out_1 = _run_round(
        kg_nodes3.reshape(B * NK, H), ke1, k1, k2, ques_kg,
        row(p['kg_att_W'][:H, 0]), row(p['kg_att_W'][2 * H:, 0]),
        p['kg_msg_W'], p['kg_agg_W'], row(p['kg_agg_v'][:, 0]), NK, EK)

    img_rel, kg_rel = _run_rel(ques_img, ques_kg, img_out_1, kg_out_1,
                               p['img_rel_W'], row(p['img_rel_b']),
                               p['kg_rel_W'], row(p['kg_rel_b']))

    _, img_out_2 = _run_round(
        img_nodes1, ie2, n1, n2, img_rel,
        row(p['imgx_att_W'][:H, 0]), row(p['imgx_att_W'][2 * H:, 0]),
        p['imgx_msg_W'], p['imgx_agg_W'], row(p['imgx_agg_v'][:, 0]), NO, EI)

    _, kg_out_2 = _run_round(
        kg_nodes1, ke2, k1, k2, kg_rel,
        row(p['kgx_att_W'][:H, 0]), row(p['kgx_att_W'][2 * H:, 0]),
        p['kgx_msg_W'], p['kgx_agg_W'], row(p['kgx_agg_v'][:, 0]), NK, EK)

    return _run_head(img_out_1, img_out_2, kg_out_1, kg_out_2,
                     p['img_gate_W'], p['kg_gate_W'], p['pred_gate_W'],
                     p['head_W'])


# speedup vs baseline: 14.9572x; 1.0070x over previous
import functools

import jax
import jax.numpy as jnp
from jax.experimental import pallas as pl
from jax.experimental.pallas import tpu as pltpu
from jax.experimental.pallas import tpu_sc as plsc

B = 32; L = 20; H = 1024; EMB = 300; VOCAB = 10000; NA = 3000
IMG = 2048; LOC = 5; NO = 36; EI = 1260; NK = 100; EK = 200; LK = 10

f32 = jnp.float32


def _dotT(a, b):
    return jax.lax.dot_general(a, b, (((1,), (1,)), ((), ())),
                               preferred_element_type=f32)


def _mm(a, b):
    return jnp.dot(a, b, preferred_element_type=f32)


_DP = 384
_GPW = 304
_CG = 8
_NCHUNK = _GPW // _CG
_NLC = (EMB + 15) // 16


def _sc_gather_body(table_ref, ids_ref, out_ref,
                    ids_all, rows0, rows1, out_v, sem0, sem1):
    wid = jax.lax.axis_index("s") * 2 + jax.lax.axis_index("c")
    pltpu.sync_copy(ids_ref.at[pl.ds(wid * _GPW * LK, _GPW * LK)], ids_all)

    def fire(c, rows_v, sem):
        pltpu.async_copy(
            table_ref.at[ids_all.at[pl.ds(c * _CG * LK, _CG * LK)]],
            rows_v, sem)

    def compute(c, rows_v, sem):
        pltpu.make_async_copy(table_ref.at[ids_all.at[:_CG * LK]],
                              rows_v, sem).wait()

        def group(g, carry2):
            for j in range(_NLC):
                sl = pl.ds(j * 16, 16)
                acc = rows_v[g * LK, sl]
                for r in range(1, LK):
                    acc = acc + rows_v[g * LK + r, sl]
                out_v[g, sl] = acc
            for j in range(_NLC, _DP // 16):
                out_v[g, pl.ds(j * 16, 16)] = jnp.zeros((16,), f32)
            return carry2

        jax.lax.fori_loop(0, _CG, group, 0)
        g0 = wid * _GPW + c * _CG
        pltpu.sync_copy(out_v, out_ref.at[pl.ds(g0, _CG)])

    fire(0, rows0, sem0)

    def pair(i, carry):
        c = i * 2
        fire(c + 1, rows1, sem1)
        compute(c, rows0, sem0)

        @pl.when(c + 2 < _NCHUNK)
        def _():
            fire(c + 2, rows0, sem0)

        compute(c + 1, rows1, sem1)
        return carry

    jax.lax.fori_loop(0, _NCHUNK // 2, pair, 0)


def _run_sc_gather(table_pad, ids_pad):
    k = pl.kernel(
        _sc_gather_body,
        mesh=plsc.VectorSubcoreMesh(core_axis_name="c", subcore_axis_name="s"),
        out_type=jax.ShapeDtypeStruct((32 * _GPW, _DP), f32),
        scratch_types=[
            pltpu.VMEM((_GPW * LK,), jnp.int32),
            pltpu.VMEM((_CG * LK, _DP), f32),
            pltpu.VMEM((_CG * LK, _DP), f32),
            pltpu.VMEM((_CG, _DP), f32),
            pltpu.SemaphoreType.DMA,
            pltpu.SemaphoreType.DMA,
        ],
    )
    return k(table_pad, ids_pad)


def _tpad_body(x_ref, o_ref):
    o_ref[:, :EMB] = x_ref[...]
    o_ref[:, EMB:] = jnp.zeros((x_ref.shape[0], _DP - EMB), f32)


def _run_table_pad(table):
    blk = 2000
    return pl.pallas_call(
        _tpad_body,
        grid=(VOCAB // blk,),
        in_specs=[pl.BlockSpec((blk, EMB), lambda i: (i, 0))],
        out_specs=pl.BlockSpec((blk, _DP), lambda i: (i, 0)),
        out_shape=jax.ShapeDtypeStruct((VOCAB, _DP), f32),
    )(table)


def _lstm_body(emb_ref, wx_ref, wh_ref, b_ref, allout_ref, hlast_ref, x_scr):
    x_scr[...] = _mm(emb_ref[...], wx_ref[...]) + b_ref[...]

    def step(t, carry):
        h, c = carry
        z = x_scr[pl.ds(t * B, B), :] + _mm(h, wh_ref[...])
        i = z[:, :H]; f = z[:, H:2 * H]; g = z[:, 2 * H:3 * H]; o = z[:, 3 * H:]
        c = jax.nn.sigmoid(f) * c + jax.nn.sigmoid(i) * jnp.tanh(g)
        h = jax.nn.sigmoid(o) * jnp.tanh(c)
        allout_ref[pl.ds(t * B, B), :] = h
        return (h, c)

    z0 = jnp.zeros((B, H), f32)
    h, _ = jax.lax.fori_loop(0, L, step, (z0, z0))
    hlast_ref[...] = h


def _run_lstm(emb_t, wx, wh, b):
    return pl.pallas_call(
        _lstm_body,
        out_shape=(jax.ShapeDtypeStruct((L * B, H), f32),
                   jax.ShapeDtypeStruct((B, H), f32)),
        scratch_shapes=[pltpu.VMEM((L * B, 4 * H), f32)],
    )(emb_t, wx, wh, b)


def _qpool_body(ao_ref, h_ref, m_ref, qiw_ref, qib_ref, qkw_ref, qkb_ref,
                w1i_ref, w2i_ref, bi_ref, w1k_ref, w2k_ref, bk_ref,
                qimg_ref, qkg_ref):
    ao = ao_ref[...]
    h = h_ref[...]
    mask = m_ref[...]

    def pool(qw, qb, w1, w2, ab):
        q = _mm(h, qw) + qb
        s = (jnp.sum(ao * w1[None], axis=-1, keepdims=True)
             + jnp.sum(q * w2, axis=-1, keepdims=True)[None]
             + ab[None])
        s = jnp.where(mask == 1, -1e32, s)
        mx = jnp.max(s, axis=0, keepdims=True)
        ex = jnp.exp(s - mx)
        a = ex / jnp.sum(ex, axis=0, keepdims=True)
        return jnp.sum(ao * a, axis=0)

    qimg_ref[...] = pool(qiw_ref[...], qib_ref[...], w1i_ref[...],
                         w2i_ref[...], bi_ref[...])
    qkg_ref[...] = pool(qkw_ref[...], qkb_ref[...], w1k_ref[...],
                        w2k_ref[...], bk_ref[...])


def _run_qpool(ao3, h, mask_t, qiw, qib, qkw, qkb, w1i, w2i, bi, w1k, w2k, bk):
    return pl.pallas_call(
        _qpool_body,
        out_shape=(jax.ShapeDtypeStruct((B, H), f32),
                   jax.ShapeDtypeStruct((B, H), f32)),
    )(ao3, h, mask_t, qiw, qib, qkw, qkb, w1i, w2i, bi, w1k, w2k, bk)


def _inode_body(feat_ref, w_ref, b_ref, loc_ref, ew_ref, eb_ref,
                wa1_ref, wa2_ref, nodes_ref, e1_ref, e2_ref):
    nodes_ref[...] = _mm(feat_ref[...], w_ref[...]) + b_ref[...]
    loc = loc_ref[...]

    def eatt(wa):
        fold = _dotT(ew_ref[...], wa)
        c = _dotT(eb_ref[...], wa)
        return jnp.sum(loc * fold, axis=0, keepdims=True) + c

    e1_ref[...] = eatt(wa1_ref[...])
    e2_ref[...] = eatt(wa2_ref[...])


def _run_inode(feat, w, b, loc_t, ew, eb, wa1, wa2):
    return pl.pallas_call(
        _inode_body,
        out_shape=(jax.ShapeDtypeStruct((B * NO, H), f32),
                   jax.ShapeDtypeStruct((1, B * EI), f32),
                   jax.ShapeDtypeStruct((1, B * EI), f32)),
    )(feat, w, b, loc_t, ew, eb, wa1, wa2)


_KGB = 16


def _kgfeat_body(ent_ref, edg_ref, eid_ref, gid_ref, nw_ref, nb_ref,
                 ew_ref, eb_ref, wa1_ref, wa2_ref,
                 nodes_ref, e1_ref, e2_ref):
    eid = eid_ref[...].reshape(_KGB * NK, LK)
    elen = jnp.maximum(jnp.sum((eid != 1.0).astype(f32), axis=-1,
                               keepdims=True), 1.0)
    efeat = ent_ref[...] / elen
    nodes_ref[...] = (_mm(efeat, nw_ref[...])
                      + nb_ref[...]).reshape(_KGB, NK, H)

    gid = gid_ref[...].reshape(_KGB * EK, LK)
    glen = jnp.maximum(jnp.sum((gid != 1.0).astype(f32), axis=-1,
                               keepdims=True), 1.0)
    gfeat = edg_ref[...] / glen

    def eatt(wa):
        fold = _dotT(ew_ref[...], wa)
        c = _dotT(eb_ref[...], wa)
        return _mm(gfeat, fold) + c

    e1_ref[...] = eatt(wa1_ref[...]).reshape(_KGB, EK, 1)
    e2_ref[...] = eatt(wa2_ref[...]).reshape(_KGB, EK, 1)


def _run_kgfeat(sums, eid, gid, nw, nb, ew, eb, wa1, wa2):
    spec = lambda shape: pl.BlockSpec((_KGB,) + shape, lambda i: (i, 0, 0))
    full = lambda a: pl.BlockSpec(a.shape, lambda i: (0,) * a.ndim)
    ent_spec = pl.BlockSpec((_KGB * NK, _DP), lambda i: (i, 0))
    edg_spec = pl.BlockSpec((_KGB * EK, _DP),
                            lambda i: (i + B * NK // (_KGB * EK), 0))
    return pl.pallas_call(
        _kgfeat_body,
        grid=(B // _KGB,),
        in_specs=[ent_spec, edg_spec, spec((NK, LK)), spec((EK, LK)),
                  full(nw), full(nb), full(ew), full(eb), full(wa1), full(wa2)],
        out_specs=(spec((NK, H)), spec((EK, 1)), spec((EK, 1))),
        out_shape=(jax.ShapeDtypeStruct((B, NK, H), f32),
                   jax.ShapeDtypeStruct((B, EK, 1), f32),
                   jax.ShapeDtypeStruct((B, EK, 1), f32)),
    )(sums, sums, eid, gid, nw, nb, ew, eb, wa1, wa2)


def _round_body(nodes_ref, eatt_ref, src_ref, dst_ref, q_ref, wn_ref, wg_ref,
                wm_ref, aw_ref, v_ref, nn_ref, out_ref, *, n, e, gb):
    nodes2 = nodes_ref[...]
    nodes3 = nodes2.reshape(gb, n, H)
    ea = eatt_ref[...]
    src = src_ref[...]
    dst = dst_ref[...]
    q = q_ref[...]
    wn = wn_ref[...]
    wg = wg_ref[...]
    n_att = jnp.sum(nodes3 * wn[None], axis=-1, keepdims=True)
    q_att = jnp.sum(q * wg, axis=-1, keepdims=True)[:, :, None]
    rows = jax.lax.broadcasted_iota(jnp.int32, (gb, n, e), 1)
    oh_src = (rows == src).astype(f32)
    oh_dst = (rows == dst).astype(f32)
    gat = jnp.sum(oh_src * n_att, axis=1, keepdims=True)
    s = jnp.tanh(gat + ea + q_att)
    mx = jnp.max(jnp.where(oh_dst > 0.5, s, -1e30), axis=2, keepdims=True)
    m_e = jnp.sum(oh_dst * mx, axis=1, keepdims=True)
    ex = jnp.exp(s - m_e)
    den = jnp.sum(oh_dst * ex, axis=2, keepdims=True)
    den_e = jnp.sum(oh_dst * den, axis=1, keepdims=True)
    alpha = ex / (den_e + 1e-9)
    adj = jax.lax.dot_general(oh_dst * alpha, oh_src,
                              (((2,), (2,)), ((0,), (0,))),
                              preferred_element_type=f32)
    aggn = jax.lax.dot_general(adj, nodes3, (((2,), (1,)), ((0,), (0,))),
                               preferred_element_type=f32)
    msg = _mm(aggn.reshape(gb * n, H), wm_ref[...])
    nn = jax.nn.relu(nodes2 + msg)
    nn_ref[...] = nn
    a1 = _mm(nn, aw_ref[:H, :])
    qw = _mm(q, aw_ref[H:, :])
    t3 = jnp.tanh(a1.reshape(gb, n, H) + qw[:, None, :])
    sc = jnp.sum(t3 * v_ref[...][None], axis=-1, keepdims=True)
    mx2 = jnp.max(sc, axis=1, keepdims=True)
    ex2 = jnp.exp(sc - mx2)
    a = ex2 / jnp.sum(ex2, axis=1, keepdims=True)
    out_ref[...] = jnp.sum(nn.reshape(gb, n, H) * a, axis=1)


def _run_round(nodes, eatt, src, dst, q, wn, wg, wm, aw, vrow, n, e):
    gb = 16
    rows = pl.BlockSpec((gb * n, H), lambda i: (i, 0))
    edges = pl.BlockSpec((gb, 1, e), lambda i: (i, 0, 0))
    qspec = pl.BlockSpec((gb, H), lambda i: (i, 0))
    full = lambda a: pl.BlockSpec(a.shape, lambda i: (0,) * a.ndim)
    return pl.pallas_call(
        functools.partial(_round_body, n=n, e=e, gb=gb),
        grid=(B // gb,),
        in_specs=[rows, edges, edges, edges, qspec,
                  full(wn), full(wg), full(wm), full(aw), full(vrow)],
        out_specs=(rows, qspec),
        out_shape=(jax.ShapeDtypeStruct((B * n, H), f32),
                   jax.ShapeDtypeStruct((B, H), f32)),
    )(nodes, eatt, src, dst, q, wn, wg, wm, aw, vrow)


def _rel_body(qi_ref, qk_ref, io_ref, ko_ref, iw_ref, ib_ref, kw_ref, kb_ref,
              ir_ref, kr_ref):
    iw = iw_ref[...]; kw = kw_ref[...]
    ir_ref[...] = (_mm(qi_ref[...], iw[:H, :]) + _mm(ko_ref[...], iw[H:, :])
                   + ib_ref[...])
    kr_ref[...] = (_mm(qk_ref[...], kw[:H, :]) + _mm(io_ref[...], kw[H:, :])
                   + kb_ref[...])


def _run_rel(qi, qk, io1, ko1, iw, ib, kw, kb):
    return pl.pallas_call(
        _rel_body,
        out_shape=(jax.ShapeDtypeStruct((B, H), f32),
                   jax.ShapeDtypeStruct((B, H), f32)),
    )(qi, qk, io1, ko1, iw, ib, kw, kb)


def _head_body(io1_ref, io2_ref, ko1_ref, ko2_ref, gi_ref, gk_ref, pg_ref,
               hw_ref, out_ref):
    img_vec = io1_ref[...] + io2_ref[...]
    kg_vec = ko1_ref[...] + ko2_ref[...]
    gate = jax.nn.sigmoid(_mm(img_vec, gi_ref[...]) + _mm(kg_vec, gk_ref[...]))
    fused = gate * img_vec + (1.0 - gate) * kg_vec
    fused = fused * jax.nn.sigmoid(_mm(fused, pg_ref[...]))
    out_ref[...] = _mm(fused, hw_ref[...])


def _run_head(io1, io2, ko1, ko2, gi, gk, pg, hw):
    return pl.pallas_call(
        _head_body,
        out_shape=jax.ShapeDtypeStruct((B, NA), f32),
    )(io1, io2, ko1, ko2, gi, gk, pg, hw)


def kernel(question, question_mask, img_feat, img_loc, img_node1_id_list,
           img_node2_id_list, kg_entity, kg_edge, kg_node1_ids_list,
           kg_node2_ids_list, params):
    p = params
    row = lambda a: a.reshape(1, -1).astype(f32)

    emb = p['word_emb'][question]
    emb_t = jnp.swapaxes(emb, 0, 1).reshape(L * B, EMB)
    all_out, h_last = _run_lstm(emb_t, p['lstm_Wx'], p['lstm_Wh'],
                                row(p['lstm_b']))
    ao3 = all_out.reshape(L, B, H)
    mask_t = jnp.swapaxes(question_mask, 0, 1)[:, :, None]
    ques_img, ques_kg = _run_qpool(
        ao3, h_last, mask_t,
        p['qi_W'], row(p['qi_b']), p['qk_W'], row(p['qk_b']),
        row(p['qia_W'][:H, 0]), row(p['qia_W'][H:, 0]), row(p['qia_b']),
        row(p['qka_W'][:H, 0]), row(p['qka_W'][H:, 0]), row(p['qka_b']))

    loc_t = jnp.swapaxes(img_loc.reshape(B * EI, LOC), 0, 1)
    img_nodes, ie1, ie2 = _run_inode(
        img_feat.reshape(B * NO, IMG), p['inode_W'], row(p['inode_b']),
        loc_t, p['iedge_W'], row(p['iedge_b']),
        row(p['img_att_W'][H:2 * H, 0]), row(p['imgx_att_W'][H:2 * H, 0]))
    ie1 = ie1.reshape(B, 1, EI)
    ie2 = ie2.reshape(B, 1, EI)

    table_pad = _run_table_pad(p['word_emb'])
    ids_pad = jnp.concatenate([
        kg_entity.reshape(-1), kg_edge.reshape(-1),
        jnp.zeros((32 * _GPW - B * (NK + EK)) * LK, kg_entity.dtype)])
    sums = _run_sc_gather(table_pad, ids_pad.astype(jnp.int32))
    nwp = jnp.pad(p['knode_W'], ((0, _DP - EMB), (0, 0)))
    ewp = jnp.pad(p['kedge_W'], ((0, _DP - EMB), (0, 0)))
    kg_nodes3, ke1, ke2 = _run_kgfeat(
        sums, kg_entity.reshape(B, NK, LK).astype(f32),
        kg_edge.reshape(B, EK, LK).astype(f32),
        nwp, row(p['knode_b']), ewp, row(p['kedge_b']),
        row(p['kg_att_W'][H:2 * H, 0]), row(p['kgx_att_W'][H:2 * H, 0]))
    ke1 = ke1.reshape(B, 1, EK)
    ke2 = ke2.reshape(B, 1, EK)

    n1 = img_node1_id_list[:, None, :]
    n2 = img_node2_id_list[:, None, :]
    k1 = kg_node1_ids_list[:, None, :]
    k2 = kg_node2_ids_list[:, None, :]

    img_nodes1, img_out_1 = _run_round(
        img_nodes, ie1, n1, n2, ques_img,
        row(p['img_att_W'][:H, 0]), row(p['img_att_W'][2 * H:, 0]),
        p['img_msg_W'], p['img_agg_W'], row(p['img_agg_v'][:, 0]), NO, EI)

    kg_nodes1, kg_out_1 = _run_round(
        kg_nodes3.reshape(B * NK, H), ke1, k1, k2, ques_kg,
        row(p['kg_att_W'][:H, 0]), row(p['kg_att_W'][2 * H:, 0]),
        p['kg_msg_W'], p['kg_agg_W'], row(p['kg_agg_v'][:, 0]), NK, EK)

    img_rel, kg_rel = _run_rel(ques_img, ques_kg, img_out_1, kg_out_1,
                               p['img_rel_W'], row(p['img_rel_b']),
                               p['kg_rel_W'], row(p['kg_rel_b']))

    _, img_out_2 = _run_round(
        img_nodes1, ie2, n1, n2, img_rel,
        row(p['imgx_att_W'][:H, 0]), row(p['imgx_att_W'][2 * H:, 0]),
        p['imgx_msg_W'], p['imgx_agg_W'], row(p['imgx_agg_v'][:, 0]), NO, EI)

    _, kg_out_2 = _run_round(
        kg_nodes1, ke2, k1, k2, kg_rel,
        row(p['kgx_att_W'][:H, 0]), row(p['kgx_att_W'][2 * H:, 0]),
        p['kgx_msg_W'], p['kgx_agg_W'], row(p['kgx_agg_v'][:, 0]), NK, EK)

    return _run_head(img_out_1, img_out_2, kg_out_1, kg_out_2,
                     p['img_gate_W'], p['kg_gate_W'], p['pred_gate_W'],
                     p['head_W'])
